# baseline (speedup 1.0000x reference)
import jax
import jax.numpy as jnp
from jax.experimental import pallas as pl
from jax.experimental.pallas import tpu as pltpu

_N = 67108864
_COLS = 8192
_ROWS = _N // _COLS
_HEAD = 8
_CHUNK_SIZES = (1016,) + (1024,) * 7
_CHUNK_STARTS = tuple(
    _HEAD + sum(_CHUNK_SIZES[:i]) for i in range(len(_CHUNK_SIZES)))
_CHUNKS = len(_CHUNK_SIZES)


def _copy_kernel(x_hbm, o_hbm, head_vmem, head_in_sem, head_out_sem, tail_sems):
    head_in = pltpu.make_async_copy(
        x_hbm.at[pl.ds(0, _HEAD), :], head_vmem, head_in_sem)
    head_in.start()
    tail_copies = []
    for i in range(_CHUNKS):
        start, size = _CHUNK_STARTS[i], _CHUNK_SIZES[i]
        c = pltpu.make_async_copy(
            x_hbm.at[pl.ds(start, size), :],
            o_hbm.at[pl.ds(start, size), :],
            tail_sems.at[i])
        c.start()
        tail_copies.append(c)

    head_in.wait()
    col = jax.lax.broadcasted_iota(jnp.int32, (1, _COLS), 1)
    head_vmem[0:1, :] = jnp.where(col == 0, jnp.float32(0.0), head_vmem[0:1, :])
    head_out = pltpu.make_async_copy(
        head_vmem, o_hbm.at[pl.ds(0, _HEAD), :], head_out_sem)
    head_out.start()
    head_out.wait()
    for c in tail_copies:
        c.wait()


def kernel(x):
    x2 = x.reshape(_ROWS, _COLS)
    out = pl.pallas_call(
        _copy_kernel,
        in_specs=[pl.BlockSpec(memory_space=pl.ANY)],
        out_specs=pl.BlockSpec(memory_space=pl.ANY),
        out_shape=jax.ShapeDtypeStruct((_ROWS, _COLS), x.dtype),
        scratch_shapes=[
            pltpu.VMEM((_HEAD, _COLS), jnp.float32),
            pltpu.SemaphoreType.DMA,
            pltpu.SemaphoreType.DMA,
            pltpu.SemaphoreType.DMA((_CHUNKS,)),
        ],
    )(x2)
    return out.reshape(_N)


# speedup vs baseline: 13.4366x; 13.4366x over previous
import jax
import jax.numpy as jnp
from jax.experimental import pallas as pl
from jax.experimental.pallas import tpu as pltpu

_N = 67108864
_COLS = 8192
_ROWS = _N // _COLS
_BLK = 256


def _copy_kernel(x_ref, o_ref):
    o_ref[...] = x_ref[...]

    @pl.when(pl.program_id(0) == 0)
    def _zero_first():
        row = x_ref[0:1, :]
        col = jax.lax.broadcasted_iota(jnp.int32, (1, _COLS), 1)
        o_ref[0:1, :] = jnp.where(col == 0, jnp.float32(0.0), row)


def kernel(x):
    x2 = x.reshape(_ROWS, _COLS)
    out = pl.pallas_call(
        _copy_kernel,
        grid=(_ROWS // _BLK,),
        in_specs=[pl.BlockSpec((_BLK, _COLS), lambda i: (i, 0))],
        out_specs=pl.BlockSpec((_BLK, _COLS), lambda i: (i, 0)),
        out_shape=jax.ShapeDtypeStruct((_ROWS, _COLS), x.dtype),
        compiler_params=pltpu.CompilerParams(
            dimension_semantics=("arbitrary",),
        ),
    )(x2)
    return out.reshape(_N)


# speedup vs baseline: 51.8633x; 3.8598x over previous
import jax
import jax.numpy as jnp
from jax.experimental import pallas as pl
from jax.experimental.pallas import tpu as pltpu

_N = 67108864
_CHUNK = 2 * 1024 * 1024


def _copy_kernel(x_ref, o_ref):
    o_ref[...] = x_ref[...]

    @pl.when(pl.program_id(0) == 0)
    def _zero_first():
        idx = jax.lax.broadcasted_iota(jnp.int32, (1024,), 0)
        o_ref[0:1024] = jnp.where(idx == 0, jnp.float32(0.0), x_ref[0:1024])


def kernel(x):
    return pl.pallas_call(
        _copy_kernel,
        grid=(_N // _CHUNK,),
        in_specs=[pl.BlockSpec((_CHUNK,), lambda i: (i,))],
        out_specs=pl.BlockSpec((_CHUNK,), lambda i: (i,)),
        out_shape=jax.ShapeDtypeStruct((_N,), x.dtype),
        compiler_params=pltpu.CompilerParams(
            dimension_semantics=("arbitrary",),
        ),
    )(x)
